# baseline (speedup 1.0000x reference)
import functools

import jax
import jax.numpy as jnp
from jax import lax
from jax.experimental import pallas as pl
from jax.experimental.pallas import tpu as pltpu
from jax.experimental.pallas import tpu_sc as plsc

N_SRC = 10000
N_DST = 10000
E = 320000
D = 128

NC = 2
NS = 16
NW = NC * NS
EPT = E // NW
K = 80
NCHUNK = EPT // K
NB = 3
NTRIP = 41
RPT = N_DST // NS
ZR = 25


def _sc_aggregate(feat_src, meta):
    mesh = plsc.VectorSubcoreMesh(core_axis_name="c", subcore_axis_name="s")

    @functools.partial(
        pl.kernel,
        out_type=(
            jax.ShapeDtypeStruct((NC, N_DST, D), jnp.float32),
            jax.ShapeDtypeStruct((NC, N_DST, 16), jnp.float32),
        ),
        mesh=mesh,
        compiler_params=pltpu.CompilerParams(use_tc_tiling_on_sc=False,
                                             needs_layout_passes=False),
        scratch_types=[
            pltpu.VMEM_SHARED((N_DST, D), jnp.float32),
            pltpu.VMEM_SHARED((N_DST, 16), jnp.float32),
            [pltpu.VMEM((K, D), jnp.float32)] * NB,
            [pltpu.VMEM((3, K), jnp.int32)] * NB,
            [pltpu.VMEM((K,), jnp.int32)] * NB,
            pltpu.VMEM((K, 16), jnp.float32),
            pltpu.VMEM((ZR, D), jnp.float32),
            pltpu.VMEM((ZR, 16), jnp.float32),
            [pltpu.SemaphoreType.DMA] * NB,
            [pltpu.SemaphoreType.DMA] * NB,
            [pltpu.SemaphoreType.DMA] * NB,
        ],
    )
    def k(feat_hbm, meta_hbm, acc_out, deg_out,
          acc_sp, deg_sp, rows_v, meta_v, dscat_v,
          e0_v, zacc_v, zdeg_v, sem_i, sem_g, sem_s):
        cid = lax.axis_index("c")
        sid = lax.axis_index("s")
        wid = sid * NC + cid
        base = wid * EPT

        def start_idx(g, b):
            off = base + g * K
            pltpu.async_copy(
                meta_hbm.at[:, pl.ds(off, K)], meta_v[b], sem_i[b])

        def wait_idx(b):
            pltpu.make_async_copy(
                meta_hbm.at[:, pl.ds(0, K)], meta_v[b], sem_i[b]).wait()

        def start_gather(b):
            pltpu.async_copy(
                feat_hbm.at[meta_v[b].at[0]], rows_v[b], sem_g[b])

        def wait_gather(b):
            pltpu.make_async_copy(
                feat_hbm.at[meta_v[b].at[0]], rows_v[b], sem_g[b]).wait()

        def start_scatter(b):
            rv = rows_v[b]

            @plsc.parallel_loop(0, K // 16, 1, unroll=2)
            def scale(g):
                wvec = plsc.bitcast(
                    meta_v[b][2, pl.ds(g * 16, 16)], jnp.float32)
                for l in range(16):
                    wl = wvec[l]
                    r = g * 16 + l
                    for j in range(D // 16):
                        rv[r, pl.ds(16 * j, 16)] = (
                            rv[r, pl.ds(16 * j, 16)] * wl)
            for q in range(K // 16):
                dscat_v[b][pl.ds(16 * q, 16)] = (
                    meta_v[b][1, pl.ds(16 * q, 16)])
            pltpu.async_copy(rv, acc_sp.at[dscat_v[b]], sem_s[b], add=True)
            pltpu.async_copy(e0_v, deg_sp.at[dscat_v[b]], sem_s[b], add=True)

        def wait_scatter(b):
            pltpu.make_async_copy(
                rows_v[b], acc_sp.at[dscat_v[b]], sem_s[b]).wait()
            pltpu.make_async_copy(
                e0_v, deg_sp.at[dscat_v[b]], sem_s[b]).wait()

        zero16 = jnp.zeros((16,), jnp.float32)
        e0 = jnp.where(lax.iota(jnp.int32, 16) == 0, 1.0, 0.0)

        def zacc_row(i, carry):
            for j in range(D // 16):
                zacc_v[i, pl.ds(16 * j, 16)] = zero16
            return carry

        lax.fori_loop(0, ZR, zacc_row, 0)

        def zdeg_row(i, carry):
            zdeg_v[i] = zero16
            return carry

        lax.fori_loop(0, ZR, zdeg_row, 0)

        def e0_row(i, carry):
            e0_v[i] = e0
            return carry

        lax.fori_loop(0, K, e0_row, 0)

        row0 = sid * RPT
        for c in range(RPT // ZR):
            pltpu.sync_copy(zacc_v, acc_sp.at[pl.ds(row0 + c * ZR, ZR)])
            pltpu.sync_copy(zdeg_v, deg_sp.at[pl.ds(row0 + c * ZR, ZR)])

        plsc.subcore_barrier()

        start_idx(0, 0)
        wait_idx(0)
        start_gather(0)
        start_idx(1, 1)
        wait_idx(1)
        start_gather(1)
        start_idx(2, 2)

        def triple(p, carry):
            for b in range(NB):
                g = 3 * p + b
                bn = (b + 2) % NB
                wait_gather(b)
                start_scatter(b)
                wait_idx(bn)

                @pl.when(g > 0)
                def _ws():
                    wait_scatter(bn)

                start_gather(bn)

                @pl.when(g + 3 < NCHUNK)
                def _pf():
                    start_idx(g + 3, b)

            return carry

        lax.fori_loop(0, NTRIP, triple, 0)
        wait_gather(0)
        start_scatter(0)
        wait_gather(1)
        start_scatter(1)
        wait_scatter(2)
        wait_scatter(0)
        wait_scatter(1)

        plsc.subcore_barrier()

        pltpu.sync_copy(acc_sp.at[pl.ds(row0, RPT)],
                        acc_out.at[cid, pl.ds(row0, RPT)])
        pltpu.sync_copy(deg_sp.at[pl.ds(row0, RPT)],
                        deg_out.at[cid, pl.ds(row0, RPT)])

    return k(feat_src, meta)


def _tc_finish(acc, deg, feat_dst, w_t, b_row):
    rows = 1000
    grid = (N_DST // rows,)

    def body(acc_ref, deg_ref, fd_ref, wt_ref, b_ref, out_ref):
        a = acc_ref[0] + acc_ref[1] + fd_ref[...]
        dg = deg_ref[0, :, 0:1] + deg_ref[1, :, 0:1] + 1.0
        h = a / dg
        out_ref[...] = (
            jnp.dot(h, wt_ref[...], preferred_element_type=jnp.float32)
            + b_ref[...])

    return pl.pallas_call(
        body,
        grid=grid,
        in_specs=[
            pl.BlockSpec((NC, rows, D), lambda i: (0, i, 0)),
            pl.BlockSpec((NC, rows, 16), lambda i: (0, i, 0)),
            pl.BlockSpec((rows, D), lambda i: (i, 0)),
            pl.BlockSpec((D, D), lambda i: (0, 0)),
            pl.BlockSpec((1, D), lambda i: (0, 0)),
        ],
        out_specs=pl.BlockSpec((rows, D), lambda i: (i, 0)),
        out_shape=jax.ShapeDtypeStruct((N_DST, D), jnp.float32),
    )(acc, deg, feat_dst, w_t, b_row)


def kernel(feat_src, feat_dst, edge_weight, edge_index, W_neigh, b_neigh):
    src = edge_index[0].astype(jnp.int32)
    dst = edge_index[1].astype(jnp.int32)
    wbits = lax.bitcast_convert_type(edge_weight.astype(jnp.float32),
                                     jnp.int32)
    meta = jnp.stack([src, dst, wbits])
    acc, deg = _sc_aggregate(feat_src, meta)
    return _tc_finish(acc, deg, feat_dst, W_neigh.T, b_neigh.reshape(1, D))


# speedup vs baseline: 1.0055x; 1.0055x over previous
import functools

import jax
import jax.numpy as jnp
from jax import lax
from jax.experimental import pallas as pl
from jax.experimental.pallas import tpu as pltpu
from jax.experimental.pallas import tpu_sc as plsc

N_SRC = 10000
N_DST = 10000
E = 320000
D = 128

NC = 2
NS = 16
NW = NC * NS
EPT = E // NW
K = 80
NCHUNK = EPT // K
NB = 3
NTRIP = 41
RPT = N_DST // NS
ZR = 25


def _sc_aggregate(feat_src, meta):
    mesh = plsc.VectorSubcoreMesh(core_axis_name="c", subcore_axis_name="s")

    @functools.partial(
        pl.kernel,
        out_type=(
            jax.ShapeDtypeStruct((NC, N_DST, D), jnp.float32),
            jax.ShapeDtypeStruct((NC, N_DST, 16), jnp.float32),
        ),
        mesh=mesh,
        compiler_params=pltpu.CompilerParams(use_tc_tiling_on_sc=False,
                                             needs_layout_passes=False),
        scratch_types=[
            pltpu.VMEM_SHARED((N_DST, D), jnp.float32),
            pltpu.VMEM_SHARED((N_DST, 16), jnp.float32),
            [pltpu.VMEM((K, D), jnp.float32)] * NB,
            [pltpu.VMEM((3, K), jnp.int32)] * NB,
            [pltpu.VMEM((K,), jnp.int32)] * NB,
            pltpu.VMEM((K, 16), jnp.float32),
            pltpu.VMEM((ZR, D), jnp.float32),
            pltpu.VMEM((ZR, 16), jnp.float32),
            [pltpu.SemaphoreType.DMA] * NB,
            [pltpu.SemaphoreType.DMA] * NB,
            [pltpu.SemaphoreType.DMA] * NB,
        ],
    )
    def k(feat_hbm, meta_hbm, acc_out, deg_out,
          acc_sp, deg_sp, rows_v, meta_v, dscat_v,
          e0_v, zacc_v, zdeg_v, sem_i, sem_g, sem_s):
        cid = lax.axis_index("c")
        sid = lax.axis_index("s")
        wid = sid * NC + cid
        base = wid * EPT

        def start_idx(g, b):
            off = base + g * K
            pltpu.async_copy(
                meta_hbm.at[:, pl.ds(off, K)], meta_v[b], sem_i[b])

        def wait_idx(b):
            pltpu.make_async_copy(
                meta_hbm.at[:, pl.ds(0, K)], meta_v[b], sem_i[b]).wait()

        def start_gather(b):
            pltpu.async_copy(
                feat_hbm.at[meta_v[b].at[0]], rows_v[b], sem_g[b])

        def wait_gather(b):
            pltpu.make_async_copy(
                feat_hbm.at[meta_v[b].at[0]], rows_v[b], sem_g[b]).wait()

        def start_scatter(b):
            rv = rows_v[b]

            def scale(g, c2):
                wvec = plsc.bitcast(
                    meta_v[b][2, pl.ds(g * 16, 16)], jnp.float32)
                for l in range(16):
                    wl = wvec[l]
                    r = g * 16 + l
                    for j in range(D // 16):
                        rv[r, pl.ds(16 * j, 16)] = (
                            rv[r, pl.ds(16 * j, 16)] * wl)
                return c2

            lax.fori_loop(0, K // 16, scale, 0)
            for q in range(K // 16):
                dscat_v[b][pl.ds(16 * q, 16)] = (
                    meta_v[b][1, pl.ds(16 * q, 16)])
            pltpu.async_copy(rv, acc_sp.at[dscat_v[b]], sem_s[b], add=True)
            pltpu.async_copy(e0_v, deg_sp.at[dscat_v[b]], sem_s[b], add=True)

        def wait_scatter(b):
            pltpu.make_async_copy(
                rows_v[b], acc_sp.at[dscat_v[b]], sem_s[b]).wait()
            pltpu.make_async_copy(
                e0_v, deg_sp.at[dscat_v[b]], sem_s[b]).wait()

        zero16 = jnp.zeros((16,), jnp.float32)
        e0 = jnp.where(lax.iota(jnp.int32, 16) == 0, 1.0, 0.0)

        def zacc_row(i, carry):
            for j in range(D // 16):
                zacc_v[i, pl.ds(16 * j, 16)] = zero16
            return carry

        lax.fori_loop(0, ZR, zacc_row, 0)

        def zdeg_row(i, carry):
            zdeg_v[i] = zero16
            return carry

        lax.fori_loop(0, ZR, zdeg_row, 0)

        def e0_row(i, carry):
            e0_v[i] = e0
            return carry

        lax.fori_loop(0, K, e0_row, 0)

        row0 = sid * RPT
        for c in range(RPT // ZR):
            pltpu.sync_copy(zacc_v, acc_sp.at[pl.ds(row0 + c * ZR, ZR)])
            pltpu.sync_copy(zdeg_v, deg_sp.at[pl.ds(row0 + c * ZR, ZR)])

        plsc.subcore_barrier()

        start_idx(0, 0)
        wait_idx(0)
        start_gather(0)
        start_idx(1, 1)
        wait_idx(1)
        start_gather(1)
        start_idx(2, 2)

        def triple(p, carry):
            for b in range(NB):
                g = 3 * p + b
                bn = (b + 2) % NB
                wait_gather(b)
                start_scatter(b)
                wait_idx(bn)

                @pl.when(g > 0)
                def _ws():
                    wait_scatter(bn)

                start_gather(bn)

                @pl.when(g + 3 < NCHUNK)
                def _pf():
                    start_idx(g + 3, b)

            return carry

        lax.fori_loop(0, NTRIP, triple, 0)
        wait_gather(0)
        start_scatter(0)
        wait_gather(1)
        start_scatter(1)
        wait_scatter(2)
        wait_scatter(0)
        wait_scatter(1)

        plsc.subcore_barrier()

        pltpu.sync_copy(acc_sp.at[pl.ds(row0, RPT)],
                        acc_out.at[cid, pl.ds(row0, RPT)])
        pltpu.sync_copy(deg_sp.at[pl.ds(row0, RPT)],
                        deg_out.at[cid, pl.ds(row0, RPT)])

    return k(feat_src, meta)


def _tc_finish(acc, deg, feat_dst, w_t, b_row):
    rows = 1000
    grid = (N_DST // rows,)

    def body(acc_ref, deg_ref, fd_ref, wt_ref, b_ref, out_ref):
        a = acc_ref[0] + acc_ref[1] + fd_ref[...]
        dg = deg_ref[0, :, 0:1] + deg_ref[1, :, 0:1] + 1.0
        h = a / dg
        out_ref[...] = (
            jnp.dot(h, wt_ref[...], preferred_element_type=jnp.float32)
            + b_ref[...])

    return pl.pallas_call(
        body,
        grid=grid,
        in_specs=[
            pl.BlockSpec((NC, rows, D), lambda i: (0, i, 0)),
            pl.BlockSpec((NC, rows, 16), lambda i: (0, i, 0)),
            pl.BlockSpec((rows, D), lambda i: (i, 0)),
            pl.BlockSpec((D, D), lambda i: (0, 0)),
            pl.BlockSpec((1, D), lambda i: (0, 0)),
        ],
        out_specs=pl.BlockSpec((rows, D), lambda i: (i, 0)),
        out_shape=jax.ShapeDtypeStruct((N_DST, D), jnp.float32),
    )(acc, deg, feat_dst, w_t, b_row)


def kernel(feat_src, feat_dst, edge_weight, edge_index, W_neigh, b_neigh):
    src = edge_index[0].astype(jnp.int32)
    dst = edge_index[1].astype(jnp.int32)
    wbits = lax.bitcast_convert_type(edge_weight.astype(jnp.float32),
                                     jnp.int32)
    meta = jnp.stack([src, dst, wbits])
    acc, deg = _sc_aggregate(feat_src, meta)
    return _tc_finish(acc, deg, feat_dst, W_neigh.T, b_neigh.reshape(1, D))
